# baseline (speedup 1.0000x reference)
import functools

import jax
import jax.numpy as jnp
from jax import lax
from jax.experimental import pallas as pl
from jax.experimental.pallas import tpu as pltpu
from jax.experimental.pallas import tpu_sc as plsc

_B = 64
_E = 8
_LANES = 16


def _sc_routing_body(logits_hbm, rn_hbm, coeff_hbm, lvm, rnvm, cvm):
    is_lead = jnp.logical_and(lax.axis_index("c") == 0,
                              lax.axis_index("s") == 0)

    @pl.when(is_lead)
    def _():
        pltpu.sync_copy(logits_hbm, lvm)
        pltpu.sync_copy(rn_hbm, rnvm)
        rn = rnvm[...]
        for chunk in range(_B // _LANES):
            sl = pl.ds(chunk * _LANES, _LANES)
            vecs = [lvm[e, sl] for e in range(_E)]
            m = vecs[0]
            for v in vecs[1:]:
                m = jnp.maximum(m, v)
            ps = [jnp.exp(v - m) for v in vecs]
            s = ps[0]
            for p in ps[1:]:
                s = s + p
            rs = [p / s for p in ps]
            m1 = rs[0]
            i1 = jnp.zeros((_LANES,), jnp.int32)
            for e in range(1, _E):
                gt = rs[e] > m1
                m1 = jnp.where(gt, rs[e], m1)
                i1 = jnp.where(gt, e, i1)
            m2 = jnp.full((_LANES,), -jnp.inf, jnp.float32)
            i2 = jnp.zeros((_LANES,), jnp.int32)
            for e in range(_E):
                cand = jnp.where(i1 == e, -jnp.inf, rs[e])
                gt = cand > m2
                m2 = jnp.where(gt, cand, m2)
                i2 = jnp.where(gt, e, i2)
            denom = m1 + m2
            w1 = jnp.where(rn != 0, m1 / denom, m1)
            w2 = jnp.where(rn != 0, m2 / denom, m2)
            for e in range(_E):
                ce = (jnp.where(i1 == e, w1, 0.0)
                      + jnp.where(i2 == e, w2, 0.0))
                cvm[e, sl] = ce
        pltpu.sync_copy(cvm, coeff_hbm)


def _routing_coeff_sc(router_logits, renormalize):
    rnvec = jnp.broadcast_to(
        jnp.asarray(renormalize, jnp.float32), (_LANES,))
    run = pl.kernel(
        _sc_routing_body,
        out_type=jax.ShapeDtypeStruct((_E, _B), jnp.float32),
        mesh=plsc.VectorSubcoreMesh(core_axis_name="c", subcore_axis_name="s"),
        scratch_types=[
            pltpu.VMEM((_E, _B), jnp.float32),
            pltpu.VMEM((_LANES,), jnp.float32),
            pltpu.VMEM((_E, _B), jnp.float32),
        ],
    )
    coeff_t = run(router_logits.astype(jnp.float32).T, rnvec)
    return coeff_t.T


def _moe_body(x_ref, coeff_ref, w1_ref, w3_ref, w2_ref, out_ref):
    e = pl.program_id(0)
    i = pl.program_id(1)

    @pl.when(jnp.logical_and(e == 0, i == 0))
    def _():
        out_ref[...] = jnp.zeros_like(out_ref)

    x = x_ref[...]
    dn = (((1,), (1,)), ((), ()))
    h1 = jax.lax.dot_general(x, w1_ref[0], dn,
                             preferred_element_type=jnp.float32)
    h3 = jax.lax.dot_general(x, w3_ref[0], dn,
                             preferred_element_type=jnp.float32)
    g = h1 * jax.nn.sigmoid(h1) * h3
    ids = jax.lax.broadcasted_iota(jnp.int32, coeff_ref.shape, 1)
    c = jnp.sum(jnp.where(ids == e, coeff_ref[...], 0.0), axis=1,
                keepdims=True)
    g = g * c
    out_ref[...] += jax.lax.dot_general(g, w2_ref[0], dn,
                                        preferred_element_type=jnp.float32)


def kernel(hidden_states, W13, W2, use_grouped_topk, top_k, router_logits,
           renormalize):
    B, H = hidden_states.shape
    num_experts, two_i, _ = W13.shape
    inter = two_i // 2
    TI = 1408
    NI = inter // TI

    coeff = _routing_coeff_sc(router_logits, renormalize)

    out = pl.pallas_call(
        _moe_body,
        grid=(num_experts, NI),
        in_specs=[
            pl.BlockSpec((B, H), lambda e, i: (0, 0)),
            pl.BlockSpec((B, num_experts), lambda e, i: (0, 0)),
            pl.BlockSpec((1, TI, H), lambda e, i: (e, i, 0)),
            pl.BlockSpec((1, TI, H), lambda e, i, ni=NI: (e, ni + i, 0)),
            pl.BlockSpec((1, H, TI), lambda e, i: (e, 0, i)),
        ],
        out_specs=pl.BlockSpec((B, H), lambda e, i: (0, 0)),
        out_shape=jax.ShapeDtypeStruct((B, H), jnp.float32),
        compiler_params=pltpu.CompilerParams(
            dimension_semantics=("arbitrary", "arbitrary")),
    )(hidden_states, coeff, W13, W13, W2)
    return out
